# baseline (speedup 1.0000x reference)
import jax
import jax.numpy as jnp
from jax.experimental import pallas as pl
from jax.experimental.pallas import tpu as pltpu


def _down_body(x_ref, w_ref, b_ref, a_ref, o32_ref, o16_ref):
    y = jnp.dot(x_ref[...], w_ref[...], preferred_element_type=jnp.float32)
    y = y + b_ref[...]
    y = jnp.where(y > 0.0, y, a_ref[...] * y)
    o32_ref[...] = y
    o16_ref[...] = y.astype(jnp.bfloat16)


def _down_conv(x_ncdhw, w_down, b_down, a_prelu, *, tile_rows=2048):
    N, Cin, D, H, W = x_ncdhw.shape
    Co = w_down.shape[0]
    D2, H2, W2 = D // 2, H // 2, W // 2

    x_cl = jnp.transpose(x_ncdhw, (0, 2, 3, 4, 1))
    xp = x_cl.reshape(N, D2, 2, H2, 2, W2, 2, Cin)
    xp = jnp.transpose(xp, (0, 1, 3, 5, 2, 4, 6, 7))
    xp = xp.reshape(N * D2 * H2 * W2, 8 * Cin).astype(jnp.bfloat16)
    w_flat = jnp.transpose(w_down, (2, 3, 4, 1, 0)).reshape(8 * Cin, Co)
    w_flat = w_flat.astype(jnp.bfloat16)
    b2 = b_down.reshape(1, Co)
    a2 = a_prelu.reshape(1, Co)

    rows = xp.shape[0]
    tm = min(tile_rows, rows)
    n_blocks = -(-rows // tm)

    cost = pl.CostEstimate(
        flops=2 * rows * 8 * Cin * Co,
        transcendentals=0,
        bytes_accessed=2 * rows * 8 * Cin + 2 * 8 * Cin * Co + 6 * rows * Co)

    y32, y16 = pl.pallas_call(
        _down_body,
        out_shape=(jax.ShapeDtypeStruct((rows, Co), jnp.float32),
                   jax.ShapeDtypeStruct((rows, Co), jnp.bfloat16)),
        grid=(n_blocks,),
        in_specs=[
            pl.BlockSpec((tm, 8 * Cin), lambda i: (i, 0)),
            pl.BlockSpec((8 * Cin, Co), lambda i: (0, 0)),
            pl.BlockSpec((1, Co), lambda i: (0, 0)),
            pl.BlockSpec((1, Co), lambda i: (0, 0)),
        ],
        out_specs=(pl.BlockSpec((tm, Co), lambda i: (i, 0)),
                   pl.BlockSpec((tm, Co), lambda i: (i, 0))),
        compiler_params=pltpu.CompilerParams(dimension_semantics=("parallel",)),
        cost_estimate=cost,
    )(xp, w_flat, b2, a2)
    return (y32.reshape(N, D2, H2, W2, Co), y16.reshape(N, D2, H2, W2, Co))


def _conv5_layer(x_ndhwc, w_oidhw, b, a_prelu, residual=None, *, td=6,
                 out_bf16=False):
    N, D, H, W, C = x_ndhwc.shape
    pad = 2
    HW = H * W
    Hp, Wp = H + 2 * pad, W + 2 * pad

    TD = td
    nD = -(-D // TD)
    Dv = nD * TD
    Dslab = TD + 4
    front = pad
    back = (nD + 1) * TD - D - front
    xpad = jnp.pad(x_ndhwc, ((0, 0), (front, back), (pad, pad), (pad, pad), (0, 0)))

    w_t = jnp.transpose(w_oidhw, (2, 3, 4, 1, 0))
    w_prep = w_t.reshape(5, 25 * C, C).astype(jnp.bfloat16)
    b2 = b.reshape(1, C)
    a2 = a_prelu.reshape(1, C)

    add_res = residual is not None
    if add_res:
        res = residual.reshape(N, D * HW, C)
        if Dv != D:
            res = jnp.pad(res, ((0, 0), (0, (Dv - D) * HW), (0, 0)))

    out_dtype = jnp.bfloat16 if out_bf16 else jnp.float32

    def _body(*refs):
        if add_res:
            xlo, xhi, w_ref, b_ref, a_ref, res_ref, o_ref = refs
        else:
            xlo, xhi, w_ref, b_ref, a_ref, o_ref = refs

        slab = jnp.concatenate([xlo[...], xhi[...]], axis=0)[:Dslab]
        pw = jnp.concatenate(
            [slab[:, :, kw:kw + W, :] for kw in range(5)], axis=-1)
        phw = jnp.concatenate(
            [pw[:, kh:kh + H] for kh in range(5)], axis=-1)
        acc = None
        for kd in range(5):
            p = phw[kd:kd + TD].reshape(TD * HW, 25 * C)
            y = jnp.dot(p, w_ref[kd], preferred_element_type=jnp.float32)
            acc = y if acc is None else acc + y
        y = acc + b_ref[...]
        y = jnp.where(y > 0.0, y, a_ref[...] * y)
        if add_res:
            y = y + res_ref[...]
        o_ref[...] = y.astype(out_dtype)

    in_specs = [
        pl.BlockSpec((None, TD, Hp, Wp, C), lambda n, d: (n, d, 0, 0, 0)),
        pl.BlockSpec((None, TD, Hp, Wp, C), lambda n, d: (n, d + 1, 0, 0, 0)),
        pl.BlockSpec(w_prep.shape, lambda n, d: (0, 0, 0)),
        pl.BlockSpec((1, C), lambda n, d: (0, 0)),
        pl.BlockSpec((1, C), lambda n, d: (0, 0)),
    ]
    inputs = [xpad, xpad, w_prep, b2, a2]
    if add_res:
        in_specs.append(pl.BlockSpec((None, TD * HW, C), lambda n, d: (n, d, 0)))
        inputs.append(res)

    cost = pl.CostEstimate(
        flops=2 * N * Dv * HW * 125 * C * C,
        transcendentals=0,
        bytes_accessed=2 * 2 * xpad.size + 2 * w_prep.size + 8 * N * Dv * HW * C)

    out = pl.pallas_call(
        _body,
        out_shape=jax.ShapeDtypeStruct((N, Dv * HW, C), out_dtype),
        grid=(N, nD),
        in_specs=in_specs,
        out_specs=pl.BlockSpec((None, TD * HW, C), lambda n, d: (n, d, 0)),
        compiler_params=pltpu.CompilerParams(
            dimension_semantics=("parallel", "parallel")),
        cost_estimate=cost,
    )(*inputs)
    return out[:, :D * HW, :]


def kernel(x, down_w, down_b, prelu1, conv_w0, conv_b0, conv_a0,
           conv_w1, conv_b1, conv_a1):
    down32, down16 = _down_conv(x, down_w, down_b, prelu1)
    N, D2, H2, W2, C = down32.shape

    h0 = _conv5_layer(down16, conv_w0, conv_b0, conv_a0, out_bf16=True)
    h0 = h0.reshape(N, D2, H2, W2, C)
    out = _conv5_layer(h0, conv_w1, conv_b1, conv_a1, residual=down32)
    out = out.reshape(N, D2, H2, W2, C)
    return jnp.transpose(out, (0, 4, 1, 2, 3))


# speedup vs baseline: 1.5912x; 1.5912x over previous
import jax
import jax.numpy as jnp
from jax.experimental import pallas as pl
from jax.experimental.pallas import tpu as pltpu


def _down_body(x_ref, w_ref, b_ref, a_ref, o32_ref, o16_ref):
    xb = x_ref[0].astype(jnp.bfloat16)
    Cin, _, H, W = xb.shape
    H2, W2 = H // 2, W // 2
    xt = jnp.transpose(xb, (1, 2, 3, 0))
    p = xt.reshape(2, H2, 2, W2, 2, Cin)
    p = jnp.transpose(p, (1, 3, 0, 2, 4, 5))
    p = p.reshape(H2 * W2, 8 * Cin)
    y = jnp.dot(p, w_ref[...], preferred_element_type=jnp.float32)
    y = y + b_ref[...]
    y = jnp.where(y > 0.0, y, a_ref[...] * y)
    o32_ref[0, 0] = y
    o16_ref[0, 0] = y.astype(jnp.bfloat16)


def _down_conv(x_ncdhw, w_down, b_down, a_prelu):
    N, Cin, D, H, W = x_ncdhw.shape
    Co = w_down.shape[0]
    D2, H2, W2 = D // 2, H // 2, W // 2

    w_flat = jnp.transpose(w_down, (2, 3, 4, 1, 0)).reshape(8 * Cin, Co)
    w_flat = w_flat.astype(jnp.bfloat16)
    b2 = b_down.reshape(1, Co)
    a2 = a_prelu.reshape(1, Co)

    rows = H2 * W2
    cost = pl.CostEstimate(
        flops=2 * N * D2 * rows * 8 * Cin * Co,
        transcendentals=0,
        bytes_accessed=4 * N * Cin * D * H * W + 6 * N * D2 * rows * Co)

    y32, y16 = pl.pallas_call(
        _down_body,
        out_shape=(jax.ShapeDtypeStruct((N, D2, rows, Co), jnp.float32),
                   jax.ShapeDtypeStruct((N, D2, rows, Co), jnp.bfloat16)),
        grid=(N, D2),
        in_specs=[
            pl.BlockSpec((1, Cin, 2, H, W), lambda n, d: (n, 0, d, 0, 0)),
            pl.BlockSpec((8 * Cin, Co), lambda n, d: (0, 0)),
            pl.BlockSpec((1, Co), lambda n, d: (0, 0)),
            pl.BlockSpec((1, Co), lambda n, d: (0, 0)),
        ],
        out_specs=(pl.BlockSpec((1, 1, rows, Co), lambda n, d: (n, d, 0, 0)),
                   pl.BlockSpec((1, 1, rows, Co), lambda n, d: (n, d, 0, 0))),
        compiler_params=pltpu.CompilerParams(
            dimension_semantics=("parallel", "parallel")),
        cost_estimate=cost,
    )(x_ncdhw, w_flat, b2, a2)
    return (y32.reshape(N, D2, H2, W2, Co), y16.reshape(N, D2, H2, W2, Co))


def _conv5_layer(x_ndhwc, w_oidhw, b, a_prelu, residual=None, *, td=6,
                 out_bf16=False):
    N, D, H, W, C = x_ndhwc.shape
    pad = 2
    HW = H * W
    Hp, Wp = H + 2 * pad, W + 2 * pad

    TD = td
    nD = -(-D // TD)
    Dv = nD * TD
    Dslab = TD + 4
    front = pad
    back = (nD + 1) * TD - D - front
    xpad = jnp.pad(x_ndhwc, ((0, 0), (front, back), (pad, pad), (pad, pad), (0, 0)))

    w_t = jnp.transpose(w_oidhw, (2, 3, 4, 1, 0))
    w_prep = w_t.reshape(5, 25 * C, C).astype(jnp.bfloat16)
    b2 = b.reshape(1, C)
    a2 = a_prelu.reshape(1, C)

    add_res = residual is not None
    if add_res:
        res = residual.reshape(N, D * HW, C)
        if Dv != D:
            res = jnp.pad(res, ((0, 0), (0, (Dv - D) * HW), (0, 0)))

    out_dtype = jnp.bfloat16 if out_bf16 else jnp.float32

    def _body(*refs):
        if add_res:
            xlo, xhi, w_ref, b_ref, a_ref, res_ref, o_ref = refs
        else:
            xlo, xhi, w_ref, b_ref, a_ref, o_ref = refs

        slab = jnp.concatenate([xlo[...], xhi[...]], axis=0)[:Dslab]
        pw = jnp.concatenate(
            [slab[:, :, kw:kw + W, :] for kw in range(5)], axis=-1)
        phw = jnp.concatenate(
            [pw[:, kh:kh + H] for kh in range(5)], axis=-1)
        acc = None
        for kd in range(5):
            p = phw[kd:kd + TD].reshape(TD * HW, 25 * C)
            y = jnp.dot(p, w_ref[kd], preferred_element_type=jnp.float32)
            acc = y if acc is None else acc + y
        y = acc + b_ref[...]
        y = jnp.where(y > 0.0, y, a_ref[...] * y)
        if add_res:
            y = y + res_ref[...]
        o_ref[...] = y.astype(out_dtype)

    in_specs = [
        pl.BlockSpec((None, TD, Hp, Wp, C), lambda n, d: (n, d, 0, 0, 0)),
        pl.BlockSpec((None, TD, Hp, Wp, C), lambda n, d: (n, d + 1, 0, 0, 0)),
        pl.BlockSpec(w_prep.shape, lambda n, d: (0, 0, 0)),
        pl.BlockSpec((1, C), lambda n, d: (0, 0)),
        pl.BlockSpec((1, C), lambda n, d: (0, 0)),
    ]
    inputs = [xpad, xpad, w_prep, b2, a2]
    if add_res:
        in_specs.append(pl.BlockSpec((None, TD * HW, C), lambda n, d: (n, d, 0)))
        inputs.append(res)

    cost = pl.CostEstimate(
        flops=2 * N * Dv * HW * 125 * C * C,
        transcendentals=0,
        bytes_accessed=2 * 2 * xpad.size + 2 * w_prep.size + 8 * N * Dv * HW * C)

    out = pl.pallas_call(
        _body,
        out_shape=jax.ShapeDtypeStruct((N, Dv * HW, C), out_dtype),
        grid=(N, nD),
        in_specs=in_specs,
        out_specs=pl.BlockSpec((None, TD * HW, C), lambda n, d: (n, d, 0)),
        compiler_params=pltpu.CompilerParams(
            dimension_semantics=("parallel", "parallel")),
        cost_estimate=cost,
    )(*inputs)
    return out[:, :D * HW, :]


def kernel(x, down_w, down_b, prelu1, conv_w0, conv_b0, conv_a0,
           conv_w1, conv_b1, conv_a1):
    down32, down16 = _down_conv(x, down_w, down_b, prelu1)
    N, D2, H2, W2, C = down32.shape

    h0 = _conv5_layer(down16, conv_w0, conv_b0, conv_a0, out_bf16=True)
    h0 = h0.reshape(N, D2, H2, W2, C)
    out = _conv5_layer(h0, conv_w1, conv_b1, conv_a1, residual=down32)
    out = out.reshape(N, D2, H2, W2, C)
    return jnp.transpose(out, (0, 4, 1, 2, 3))


# speedup vs baseline: 1.8625x; 1.1705x over previous
import jax
import jax.numpy as jnp
from jax.experimental import pallas as pl
from jax.experimental.pallas import tpu as pltpu


def _down_body(x_ref, w_ref, b_ref, a_ref, m_ref, o32_ref, o16_ref):
    xb = x_ref[0].astype(jnp.bfloat16)
    Cin, _, H, W = xb.shape
    H2 = H // 2
    WB = o16_ref.shape[-2]
    L = m_ref.shape[-1]
    xt = jnp.transpose(xb, (1, 2, 3, 0))
    sl = [xt[:, :, min(max(8 * (wb - 1), 0), W - 8):, :][:, :, :8, :]
          for wb in range(WB)]
    ps = jnp.stack(sl, axis=0)
    ps = ps.reshape(WB, 2, H2, 2, 8, Cin)
    p = jnp.transpose(ps, (2, 0, 1, 3, 4, 5))
    p = p.reshape(H2 * WB, 32 * Cin)
    y = jnp.dot(p, w_ref[...], preferred_element_type=jnp.float32)
    y = y + b_ref[...]
    y = jnp.where(y > 0.0, y, a_ref[...] * y)
    y3 = y.reshape(H2, WB, L) * m_ref[...]
    zh = jnp.zeros((2, WB, L), jnp.float32)
    yf = jnp.concatenate([zh, y3, zh], axis=0)
    o32_ref[0] = yf
    o16_ref[0] = yf.astype(jnp.bfloat16)


def _down_conv(x_ncdhw, w_down, b_down, a_prelu):
    N, Cin, D, H, W = x_ncdhw.shape
    Co = w_down.shape[0]
    D2, H2, W2 = D // 2, H // 2, W // 2

    wt = jnp.transpose(w_down, (2, 3, 4, 1, 0))
    wd6 = jnp.zeros((2, 2, 8, Cin, 4, Co), wt.dtype)
    for ws in range(4):
        wd6 = wd6.at[:, :, 2 * ws:2 * ws + 2, :, ws, :].set(wt)
    w_band = wd6.reshape(32 * Cin, 4 * Co).astype(jnp.bfloat16)
    b128 = jnp.tile(b_down, 4).reshape(1, 4 * Co)
    a128 = jnp.tile(a_prelu, 4).reshape(1, 4 * Co)

    WB = (W2 + 8) // 4
    wpos = jnp.arange(WB)[:, None] * 4 + jnp.arange(4 * Co)[None, :] // Co - 4
    mask = ((wpos >= 0) & (wpos < W2)).astype(jnp.float32)

    rows = H2 * W2
    cost = pl.CostEstimate(
        flops=2 * N * D2 * H2 * WB * 32 * Cin * 4 * Co,
        transcendentals=0,
        bytes_accessed=4 * N * Cin * D * H * W + 8 * N * D2 * rows * Co)

    pf = (H2 + 4, WB, 4 * Co)
    y32, y16 = pl.pallas_call(
        _down_body,
        out_shape=(jax.ShapeDtypeStruct((N, D2) + pf, jnp.float32),
                   jax.ShapeDtypeStruct((N, D2) + pf, jnp.bfloat16)),
        grid=(N, D2),
        in_specs=[
            pl.BlockSpec((1, Cin, 2, H, W), lambda n, d: (n, 0, d, 0, 0)),
            pl.BlockSpec((32 * Cin, 4 * Co), lambda n, d: (0, 0)),
            pl.BlockSpec((1, 4 * Co), lambda n, d: (0, 0)),
            pl.BlockSpec((1, 4 * Co), lambda n, d: (0, 0)),
            pl.BlockSpec((WB, 4 * Co), lambda n, d: (0, 0)),
        ],
        out_specs=(pl.BlockSpec((None, 1) + pf, lambda n, d: (n, d, 0, 0, 0)),
                   pl.BlockSpec((None, 1) + pf, lambda n, d: (n, d, 0, 0, 0))),
        compiler_params=pltpu.CompilerParams(
            dimension_semantics=("parallel", "parallel")),
        cost_estimate=cost,
    )(x_ncdhw, w_band, b128, a128, mask)
    return y32, y16


def _prep_conv_w(w_oidhw):
    wt = jnp.transpose(w_oidhw, (3, 4, 1, 2, 0))
    C = wt.shape[-1]
    w6 = jnp.zeros((5, 12, C, 5, 4, C), wt.dtype)
    for ws in range(4):
        w6 = w6.at[:, ws + 2:ws + 7, :, :, ws, :].set(wt)
    return w6.reshape(5, 12 * C, 5 * 4 * C).astype(jnp.bfloat16)


def _make_conv_body(TD, nD, H2, W2, WB, last):
    def _body(*refs):
        if last:
            xm1, x0, xp1, w_ref, b_ref, a_ref, m_ref, res_ref, o_ref = refs
        else:
            xm1, x0, xp1, w_ref, b_ref, a_ref, m_ref, o_ref = refs
        C = m_ref.shape[-1] // 4
        db = pl.program_id(1)
        slab = jnp.concatenate([xm1[...], x0[...], xp1[...]], axis=0)
        slab = slab[TD - 2:2 * TD + 2]
        din = jax.lax.broadcasted_iota(jnp.int32, (TD + 4, 1, 1, 1), 0) \
            + TD * db - 2
        slab = jnp.where((din >= 0) & (din < TD * nD), slab, jnp.bfloat16(0))

        z = jnp.zeros_like(slab[:, :, :1])
        left = jnp.concatenate([z, slab[:, :, :-1]], axis=2)
        right = jnp.concatenate([slab[:, :, 1:], z], axis=2)
        pw = jnp.concatenate([left, slab, right], axis=-1)

        acc = None
        for kh in range(5):
            p = pw[:, kh:kh + H2].reshape((TD + 4) * H2 * WB, 12 * C)
            y = jnp.dot(p, w_ref[kh], preferred_element_type=jnp.float32)
            acc = y if acc is None else acc + y
        y4 = acc.reshape(TD + 4, H2, WB, 20 * C)

        out = None
        for kd in range(5):
            t = y4[kd:kd + TD, :, :, 4 * C * kd:4 * C * (kd + 1)]
            out = t if out is None else out + t
        out = out + b_ref[...]
        out = jnp.where(out > 0.0, out, a_ref[...] * out)
        out = out * m_ref[...]

        zh = jnp.zeros((TD, 2, WB, 4 * C), out.dtype)
        out = jnp.concatenate([zh, out, zh], axis=1)
        if last:
            o_ref[...] = out + res_ref[...]
        else:
            o_ref[...] = out.astype(jnp.bfloat16)
    return _body


def _conv5_layer(x_pf, w_oidhw, b, a_prelu, residual=None, *, td=6):
    N, D, Hp, WB = x_pf.shape[:4]
    H2 = Hp - 4
    C = w_oidhw.shape[0]
    W2 = WB * 4 - 8
    TD = td
    nD = D // TD
    last = residual is not None

    w_prep = _prep_conv_w(w_oidhw)
    b128 = jnp.tile(b, 4).reshape(1, 4 * C)
    a128 = jnp.tile(a_prelu, 4).reshape(1, 4 * C)
    wpos = jnp.arange(WB)[:, None] * 4 + jnp.arange(4 * C)[None, :] // C - 4
    mask = ((wpos >= 0) & (wpos < W2)).astype(jnp.float32)

    in_specs = [
        pl.BlockSpec((None, TD, Hp, WB, 4 * C),
                     lambda n, d: (n, jnp.maximum(d - 1, 0), 0, 0, 0)),
        pl.BlockSpec((None, TD, Hp, WB, 4 * C), lambda n, d: (n, d, 0, 0, 0)),
        pl.BlockSpec((None, TD, Hp, WB, 4 * C),
                     lambda n, d: (n, jnp.minimum(d + 1, nD - 1), 0, 0, 0)),
        pl.BlockSpec(w_prep.shape, lambda n, d: (0, 0, 0)),
        pl.BlockSpec((1, 4 * C), lambda n, d: (0, 0)),
        pl.BlockSpec((1, 4 * C), lambda n, d: (0, 0)),
        pl.BlockSpec((WB, 4 * C), lambda n, d: (0, 0)),
    ]
    inputs = [x_pf, x_pf, x_pf, w_prep, b128, a128, mask]
    out_spec = pl.BlockSpec((None, TD, Hp, WB, 4 * C),
                            lambda n, d: (n, d, 0, 0, 0))
    if last:
        in_specs.append(pl.BlockSpec((None, TD, Hp, WB, 4 * C),
                                     lambda n, d: (n, d, 0, 0, 0)))
        inputs.append(residual)
        out_shape = jax.ShapeDtypeStruct((N, D, Hp, WB, 4 * C), jnp.float32)
    else:
        out_shape = jax.ShapeDtypeStruct((N, D, Hp, WB, 4 * C), jnp.bfloat16)

    cost = pl.CostEstimate(
        flops=2 * N * D * H2 * W2 * 125 * C * C,
        transcendentals=0,
        bytes_accessed=3 * x_pf.size * 2 + 2 * w_prep.size
        + (8 if last else 2) * N * D * H2 * W2 * C)

    return pl.pallas_call(
        _make_conv_body(TD, nD, H2, W2, WB, last),
        out_shape=out_shape,
        grid=(N, nD),
        in_specs=in_specs,
        out_specs=out_spec,
        compiler_params=pltpu.CompilerParams(
            dimension_semantics=("parallel", "parallel")),
        cost_estimate=cost,
    )(*inputs)


def kernel(x, down_w, down_b, prelu1, conv_w0, conv_b0, conv_a0,
           conv_w1, conv_b1, conv_a1):
    res32, down16 = _down_conv(x, down_w, down_b, prelu1)
    N, D2 = down16.shape[:2]
    C = down_w.shape[0]
    H2, W2 = x.shape[3] // 2, x.shape[4] // 2

    h0 = _conv5_layer(down16, conv_w0, conv_b0, conv_a0)
    out = _conv5_layer(h0, conv_w1, conv_b1, conv_a1, residual=res32)
    WB = out.shape[3]
    out = out.reshape(N, D2, H2 + 4, 4 * WB, C)[:, :, 2:2 + H2, 4:4 + W2, :]
    return jnp.transpose(out, (0, 4, 1, 2, 3))


# speedup vs baseline: 1.9099x; 1.0255x over previous
import jax
import jax.numpy as jnp
from jax.experimental import pallas as pl
from jax.experimental.pallas import tpu as pltpu


def _down_body(x_ref, w_ref, b_ref, a_ref, m_ref, o32_ref, o16_ref):
    xb = x_ref[0].astype(jnp.bfloat16)
    Cin, D8, H, W = xb.shape
    DP, H2 = D8 // 2, H // 2
    WB = o16_ref.shape[-2]
    L = m_ref.shape[-1]
    xt = jnp.transpose(xb, (1, 2, 3, 0))
    sl = [xt[:, :, min(max(8 * (wb - 1), 0), W - 8):, :][:, :, :8, :]
          for wb in range(WB)]
    ps = jnp.stack(sl, axis=0)
    ps = ps.reshape(WB, DP, 2, H2, 2, 8, Cin)
    p = jnp.transpose(ps, (1, 3, 0, 2, 4, 5, 6))
    p = p.reshape(DP * H2 * WB, 32 * Cin)
    y = jnp.dot(p, w_ref[...], preferred_element_type=jnp.float32)
    y = y + b_ref[...]
    y = jnp.where(y > 0.0, y, a_ref[...] * y)
    y4 = y.reshape(DP, H2, WB, L) * m_ref[...]
    zh = jnp.zeros((DP, 2, WB, L), jnp.float32)
    yf = jnp.concatenate([zh, y4, zh], axis=1)
    o32_ref[...] = yf
    o16_ref[...] = yf.astype(jnp.bfloat16)


def _down_conv(x_ncdhw, w_down, b_down, a_prelu):
    N, Cin, D, H, W = x_ncdhw.shape
    Co = w_down.shape[0]
    D2, H2, W2 = D // 2, H // 2, W // 2

    wt = jnp.transpose(w_down, (2, 3, 4, 1, 0))
    wd6 = jnp.zeros((2, 2, 8, Cin, 4, Co), wt.dtype)
    for ws in range(4):
        wd6 = wd6.at[:, :, 2 * ws:2 * ws + 2, :, ws, :].set(wt)
    w_band = wd6.reshape(32 * Cin, 4 * Co).astype(jnp.bfloat16)
    b128 = jnp.tile(b_down, 4).reshape(1, 4 * Co)
    a128 = jnp.tile(a_prelu, 4).reshape(1, 4 * Co)

    WB = (W2 + 8) // 4
    wpos = jnp.arange(WB)[:, None] * 4 + jnp.arange(4 * Co)[None, :] // Co - 4
    mask = ((wpos >= 0) & (wpos < W2)).astype(jnp.float32)

    rows = H2 * W2
    cost = pl.CostEstimate(
        flops=2 * N * D2 * H2 * WB * 32 * Cin * 4 * Co,
        transcendentals=0,
        bytes_accessed=4 * N * Cin * D * H * W + 8 * N * D2 * rows * Co)

    DP = 4 if D2 % 4 == 0 else 1
    pf = (H2 + 4, WB, 4 * Co)
    y32, y16 = pl.pallas_call(
        _down_body,
        out_shape=(jax.ShapeDtypeStruct((N, D2) + pf, jnp.float32),
                   jax.ShapeDtypeStruct((N, D2) + pf, jnp.bfloat16)),
        grid=(N, D2 // DP),
        in_specs=[
            pl.BlockSpec((1, Cin, 2 * DP, H, W), lambda n, d: (n, 0, d, 0, 0)),
            pl.BlockSpec((32 * Cin, 4 * Co), lambda n, d: (0, 0)),
            pl.BlockSpec((1, 4 * Co), lambda n, d: (0, 0)),
            pl.BlockSpec((1, 4 * Co), lambda n, d: (0, 0)),
            pl.BlockSpec((WB, 4 * Co), lambda n, d: (0, 0)),
        ],
        out_specs=(pl.BlockSpec((None, DP) + pf,
                                lambda n, d: (n, d, 0, 0, 0)),
                   pl.BlockSpec((None, DP) + pf,
                                lambda n, d: (n, d, 0, 0, 0))),
        compiler_params=pltpu.CompilerParams(
            dimension_semantics=("parallel", "parallel")),
        cost_estimate=cost,
    )(x_ncdhw, w_band, b128, a128, mask)
    return y32, y16


def _prep_conv_w(w_oidhw):
    wt = jnp.transpose(w_oidhw, (3, 4, 1, 2, 0))
    C = wt.shape[-1]
    w6 = jnp.zeros((5, 12, C, 5, 4, C), wt.dtype)
    for ws in range(4):
        w6 = w6.at[:, ws + 2:ws + 7, :, :, ws, :].set(wt)
    return w6.reshape(5, 12 * C, 5 * 4 * C).astype(jnp.bfloat16)


def _make_conv_body(TD, nD, H2, W2, WB, last):
    def _body(*refs):
        if last:
            xm1, x0, xp1, w_ref, b_ref, a_ref, m_ref, res_ref, o_ref = refs
        else:
            xm1, x0, xp1, w_ref, b_ref, a_ref, m_ref, o_ref = refs
        C = m_ref.shape[-1] // 4
        db = pl.program_id(1)
        slab = jnp.concatenate([xm1[...], x0[...], xp1[...]], axis=0)
        slab = slab[TD - 2:2 * TD + 2]
        din = jax.lax.broadcasted_iota(jnp.int32, (TD + 4, 1, 1, 1), 0) \
            + TD * db - 2
        slab = jnp.where((din >= 0) & (din < TD * nD), slab, jnp.bfloat16(0))

        z = jnp.zeros_like(slab[:, :, :1])
        left = jnp.concatenate([z, slab[:, :, :-1]], axis=2)
        right = jnp.concatenate([slab[:, :, 1:], z], axis=2)
        pw = jnp.concatenate([left, slab, right], axis=-1)

        acc = None
        for kh in range(5):
            p = pw[:, kh:kh + H2].reshape((TD + 4) * H2 * WB, 12 * C)
            y = jnp.dot(p, w_ref[kh], preferred_element_type=jnp.float32)
            acc = y if acc is None else acc + y
        y4 = acc.reshape(TD + 4, H2, WB, 20 * C)

        out = None
        for kd in range(5):
            t = y4[kd:kd + TD, :, :, 4 * C * kd:4 * C * (kd + 1)]
            out = t if out is None else out + t
        out = out + b_ref[...]
        out = jnp.where(out > 0.0, out, a_ref[...] * out)
        out = out * m_ref[...]

        zh = jnp.zeros((TD, 2, WB, 4 * C), out.dtype)
        out = jnp.concatenate([zh, out, zh], axis=1)
        if last:
            o_ref[...] = out + res_ref[...]
        else:
            o_ref[...] = out.astype(jnp.bfloat16)
    return _body


def _conv5_layer(x_pf, w_oidhw, b, a_prelu, residual=None, *, td=6):
    N, D, Hp, WB = x_pf.shape[:4]
    H2 = Hp - 4
    C = w_oidhw.shape[0]
    W2 = WB * 4 - 8
    TD = td
    nD = D // TD
    last = residual is not None

    w_prep = _prep_conv_w(w_oidhw)
    b128 = jnp.tile(b, 4).reshape(1, 4 * C)
    a128 = jnp.tile(a_prelu, 4).reshape(1, 4 * C)
    wpos = jnp.arange(WB)[:, None] * 4 + jnp.arange(4 * C)[None, :] // C - 4
    mask = ((wpos >= 0) & (wpos < W2)).astype(jnp.float32)

    in_specs = [
        pl.BlockSpec((None, TD, Hp, WB, 4 * C),
                     lambda n, d: (n, jnp.maximum(d - 1, 0), 0, 0, 0)),
        pl.BlockSpec((None, TD, Hp, WB, 4 * C), lambda n, d: (n, d, 0, 0, 0)),
        pl.BlockSpec((None, TD, Hp, WB, 4 * C),
                     lambda n, d: (n, jnp.minimum(d + 1, nD - 1), 0, 0, 0)),
        pl.BlockSpec(w_prep.shape, lambda n, d: (0, 0, 0)),
        pl.BlockSpec((1, 4 * C), lambda n, d: (0, 0)),
        pl.BlockSpec((1, 4 * C), lambda n, d: (0, 0)),
        pl.BlockSpec((WB, 4 * C), lambda n, d: (0, 0)),
    ]
    inputs = [x_pf, x_pf, x_pf, w_prep, b128, a128, mask]
    out_spec = pl.BlockSpec((None, TD, Hp, WB, 4 * C),
                            lambda n, d: (n, d, 0, 0, 0))
    if last:
        in_specs.append(pl.BlockSpec((None, TD, Hp, WB, 4 * C),
                                     lambda n, d: (n, d, 0, 0, 0)))
        inputs.append(residual)
        out_shape = jax.ShapeDtypeStruct((N, D, Hp, WB, 4 * C), jnp.float32)
    else:
        out_shape = jax.ShapeDtypeStruct((N, D, Hp, WB, 4 * C), jnp.bfloat16)

    cost = pl.CostEstimate(
        flops=2 * N * D * H2 * W2 * 125 * C * C,
        transcendentals=0,
        bytes_accessed=3 * x_pf.size * 2 + 2 * w_prep.size
        + (8 if last else 2) * N * D * H2 * W2 * C)

    return pl.pallas_call(
        _make_conv_body(TD, nD, H2, W2, WB, last),
        out_shape=out_shape,
        grid=(N, nD),
        in_specs=in_specs,
        out_specs=out_spec,
        compiler_params=pltpu.CompilerParams(
            dimension_semantics=("parallel", "parallel")),
        cost_estimate=cost,
    )(*inputs)


def kernel(x, down_w, down_b, prelu1, conv_w0, conv_b0, conv_a0,
           conv_w1, conv_b1, conv_a1):
    res32, down16 = _down_conv(x, down_w, down_b, prelu1)
    N, D2 = down16.shape[:2]
    C = down_w.shape[0]
    H2, W2 = x.shape[3] // 2, x.shape[4] // 2

    h0 = _conv5_layer(down16, conv_w0, conv_b0, conv_a0)
    out = _conv5_layer(h0, conv_w1, conv_b1, conv_a1, residual=res32)
    WB = out.shape[3]
    out = out.reshape(N, D2, H2 + 4, 4 * WB, C)[:, :, 2:2 + H2, 4:4 + W2, :]
    return jnp.transpose(out, (0, 4, 1, 2, 3))


# speedup vs baseline: 2.0658x; 1.0816x over previous
import jax
import jax.numpy as jnp
from jax.experimental import pallas as pl
from jax.experimental.pallas import tpu as pltpu


def _down_body(x_ref, w_ref, b_ref, a_ref, m_ref, o32_ref, o16_ref):
    Cin, D, H, W = x_ref.shape[1:]
    D2, H2 = D // 2, H // 2
    WB = o16_ref.shape[-2]
    L = m_ref.shape[-1]
    TDo = D2 if D2 <= 8 else (D2 // 4 if D2 % 4 == 0 else D2)
    for ck in range(D2 // TDo):
        xb = x_ref[0, :, 2 * TDo * ck:2 * TDo * (ck + 1)].astype(jnp.bfloat16)
        xt = jnp.transpose(xb, (1, 2, 3, 0))
        sl = [xt[:, :, min(max(8 * (wb - 1), 0), W - 8):, :][:, :, :8, :]
              for wb in range(WB)]
        ps = jnp.stack(sl, axis=0)
        ps = ps.reshape(WB, TDo, 2, H2, 2, 8, Cin)
        p = jnp.transpose(ps, (1, 3, 0, 2, 4, 5, 6))
        p = p.reshape(TDo * H2 * WB, 32 * Cin)
        y = jnp.dot(p, w_ref[...], preferred_element_type=jnp.float32)
        y = y + b_ref[...]
        y = jnp.where(y > 0.0, y, a_ref[...] * y)
        y4 = y.reshape(TDo, H2, WB, L) * m_ref[...]
        zh = jnp.zeros((TDo, 2, WB, L), jnp.float32)
        yf = jnp.concatenate([zh, y4, zh], axis=1)
        o32_ref[TDo * ck:TDo * (ck + 1)] = yf
        o16_ref[TDo * ck:TDo * (ck + 1)] = yf.astype(jnp.bfloat16)


def _down_conv(x_ncdhw, w_down, b_down, a_prelu):
    N, Cin, D, H, W = x_ncdhw.shape
    Co = w_down.shape[0]
    D2, H2, W2 = D // 2, H // 2, W // 2

    wt = jnp.transpose(w_down, (2, 3, 4, 1, 0))
    wd6 = jnp.zeros((2, 2, 8, Cin, 4, Co), wt.dtype)
    for ws in range(4):
        wd6 = wd6.at[:, :, 2 * ws:2 * ws + 2, :, ws, :].set(wt)
    w_band = wd6.reshape(32 * Cin, 4 * Co).astype(jnp.bfloat16)
    b128 = jnp.tile(b_down, 4).reshape(1, 4 * Co)
    a128 = jnp.tile(a_prelu, 4).reshape(1, 4 * Co)

    WB = (W2 + 8) // 4
    wpos = jnp.arange(WB)[:, None] * 4 + jnp.arange(4 * Co)[None, :] // Co - 4
    mask = ((wpos >= 0) & (wpos < W2)).astype(jnp.float32)

    rows = H2 * W2
    cost = pl.CostEstimate(
        flops=2 * N * D2 * H2 * WB * 32 * Cin * 4 * Co,
        transcendentals=0,
        bytes_accessed=4 * N * Cin * D * H * W + 8 * N * D2 * rows * Co)

    DP = 4 if D2 % 4 == 0 else 1
    pf = (H2 + 4, WB, 4 * Co)
    y32, y16 = pl.pallas_call(
        _down_body,
        out_shape=(jax.ShapeDtypeStruct((N, D2) + pf, jnp.float32),
                   jax.ShapeDtypeStruct((N, D2) + pf, jnp.bfloat16)),
        grid=(N, D2 // DP),
        in_specs=[
            pl.BlockSpec((1, Cin, 2 * DP, H, W), lambda n, d: (n, 0, d, 0, 0)),
            pl.BlockSpec((32 * Cin, 4 * Co), lambda n, d: (0, 0)),
            pl.BlockSpec((1, 4 * Co), lambda n, d: (0, 0)),
            pl.BlockSpec((1, 4 * Co), lambda n, d: (0, 0)),
            pl.BlockSpec((WB, 4 * Co), lambda n, d: (0, 0)),
        ],
        out_specs=(pl.BlockSpec((None, DP) + pf,
                                lambda n, d: (n, d, 0, 0, 0)),
                   pl.BlockSpec((None, DP) + pf,
                                lambda n, d: (n, d, 0, 0, 0))),
        compiler_params=pltpu.CompilerParams(
            dimension_semantics=("parallel", "parallel")),
        cost_estimate=cost,
    )(x_ncdhw, w_band, b128, a128, mask)
    return y32, y16


def _prep_conv_w(w_oidhw):
    wt = jnp.transpose(w_oidhw, (3, 4, 1, 2, 0))
    C = wt.shape[-1]
    w6 = jnp.zeros((5, 12, C, 5, 4, C), wt.dtype)
    for ws in range(4):
        w6 = w6.at[:, ws + 2:ws + 7, :, :, ws, :].set(wt)
    return w6.reshape(5, 12 * C, 5 * 4 * C).astype(jnp.bfloat16)


def _make_conv_body(D2, H2, W2, WB, last):
    def _body(*refs):
        if last:
            x0, w_ref, b_ref, a_ref, m_ref, res_ref, o_ref = refs
        else:
            x0, w_ref, b_ref, a_ref, m_ref, o_ref = refs
        C = m_ref.shape[-1] // 4
        xv = x0[...]
        zD = jnp.zeros((2,) + xv.shape[1:], xv.dtype)
        slab = jnp.concatenate([zD, xv, zD], axis=0)

        z = jnp.zeros_like(slab[:, :, :1])
        left = jnp.concatenate([z, slab[:, :, :-1]], axis=2)
        right = jnp.concatenate([slab[:, :, 1:], z], axis=2)
        pw = jnp.concatenate([left, slab, right], axis=-1)

        TDc = D2 // 2 if D2 % 2 == 0 else D2
        for ck in range(D2 // TDc):
            pwc = pw[TDc * ck:TDc * ck + TDc + 4]
            acc = None
            for kh in range(5):
                p = pwc[:, kh:kh + H2].reshape((TDc + 4) * H2 * WB, 12 * C)
                y = jnp.dot(p, w_ref[kh], preferred_element_type=jnp.float32)
                acc = y if acc is None else acc + y
            y4 = acc.reshape(TDc + 4, H2, WB, 20 * C)

            out = None
            for kd in range(5):
                t = y4[kd:kd + TDc, :, :, 4 * C * kd:4 * C * (kd + 1)]
                out = t if out is None else out + t
            out = out + b_ref[...]
            out = jnp.where(out > 0.0, out, a_ref[...] * out)
            out = out * m_ref[...]

            zh = jnp.zeros((TDc, 2, WB, 4 * C), out.dtype)
            out = jnp.concatenate([zh, out, zh], axis=1)
            sel = slice(TDc * ck, TDc * (ck + 1))
            if last:
                o_ref[sel] = out + res_ref[sel]
            else:
                o_ref[sel] = out.astype(jnp.bfloat16)
    return _body


def _conv5_layer(x_pf, w_oidhw, b, a_prelu, residual=None):
    N, D2, Hp, WB = x_pf.shape[:4]
    H2 = Hp - 4
    C = w_oidhw.shape[0]
    W2 = WB * 4 - 8
    last = residual is not None

    w_prep = _prep_conv_w(w_oidhw)
    b128 = jnp.tile(b, 4).reshape(1, 4 * C)
    a128 = jnp.tile(a_prelu, 4).reshape(1, 4 * C)
    wpos = jnp.arange(WB)[:, None] * 4 + jnp.arange(4 * C)[None, :] // C - 4
    mask = ((wpos >= 0) & (wpos < W2)).astype(jnp.float32)

    vol = pl.BlockSpec((None, D2, Hp, WB, 4 * C), lambda n: (n, 0, 0, 0, 0))
    in_specs = [
        vol,
        pl.BlockSpec(w_prep.shape, lambda n: (0, 0, 0)),
        pl.BlockSpec((1, 4 * C), lambda n: (0, 0)),
        pl.BlockSpec((1, 4 * C), lambda n: (0, 0)),
        pl.BlockSpec((WB, 4 * C), lambda n: (0, 0)),
    ]
    inputs = [x_pf, w_prep, b128, a128, mask]
    if last:
        in_specs.append(vol)
        inputs.append(residual)
        out_shape = jax.ShapeDtypeStruct((N, D2, Hp, WB, 4 * C), jnp.float32)
    else:
        out_shape = jax.ShapeDtypeStruct((N, D2, Hp, WB, 4 * C), jnp.bfloat16)

    cost = pl.CostEstimate(
        flops=2 * N * D2 * H2 * W2 * 125 * C * C,
        transcendentals=0,
        bytes_accessed=x_pf.size * 2 + 2 * w_prep.size
        + (8 if last else 2) * N * D2 * H2 * W2 * C)

    return pl.pallas_call(
        _make_conv_body(D2, H2, W2, WB, last),
        out_shape=out_shape,
        grid=(N,),
        in_specs=in_specs,
        out_specs=vol,
        compiler_params=pltpu.CompilerParams(
            dimension_semantics=("parallel",)),
        cost_estimate=cost,
    )(*inputs)


def kernel(x, down_w, down_b, prelu1, conv_w0, conv_b0, conv_a0,
           conv_w1, conv_b1, conv_a1):
    res32, down16 = _down_conv(x, down_w, down_b, prelu1)
    N, D2 = down16.shape[:2]
    C = down_w.shape[0]
    H2, W2 = x.shape[3] // 2, x.shape[4] // 2

    h0 = _conv5_layer(down16, conv_w0, conv_b0, conv_a0)
    out = _conv5_layer(h0, conv_w1, conv_b1, conv_a1, residual=res32)
    WB = out.shape[3]
    out = out.reshape(N, D2, H2 + 4, 4 * WB, C)[:, :, 2:2 + H2, 4:4 + W2, :]
    return jnp.transpose(out, (0, 4, 1, 2, 3))
